# baseline (speedup 1.0000x reference)
import functools

import jax
import jax.numpy as jnp
from jax import lax
from jax.experimental import pallas as pl
from jax.experimental.pallas import tpu as pltpu
from jax.experimental.pallas import tpu_sc as plsc

N = 10000
E = 320000
D = 128
BN_EPS = 1e-5

NC = 2
NS = 16
NW = NC * NS

CHUNK = 128
NSET = 3
TPT = 81
E_PAD = NW * TPT * CHUNK
ACC_ROWS = N + 8

RPT = 624
RPT_LAST = N - 15 * RPT


def _seg_sum_body(h_hbm, srcb_hbm, dstb_hbm, zeros_hbm, out_hbm,
                  si0, si1, si2, di0, di1, di2, r0, r1, r2, acc,
                  sem_i, sem_g, sem_s):
    cid = lax.axis_index("c")
    sid = lax.axis_index("s")
    wid = cid * NS + sid
    sidx = [si0, si1, si2]
    didx = [di0, di1, di2]
    rows = [r0, r1, r2]

    def idx_issue(j, s):
        pltpu.async_copy(srcb_hbm.at[wid, j], sidx[s], sem_i)
        pltpu.async_copy(dstb_hbm.at[wid, j], didx[s], sem_i)

    def idx_drain(s):
        pltpu.make_async_copy(dstb_hbm.at[wid, 0], sidx[s], sem_i).wait()
        pltpu.make_async_copy(dstb_hbm.at[wid, 0], didx[s], sem_i).wait()

    def gather_issue(s):
        pltpu.async_copy(h_hbm.at[sidx[s]], rows[s], sem_g)

    def gather_drain(s):
        pltpu.make_async_copy(zeros_hbm.at[pl.ds(0, CHUNK)], rows[s],
                              sem_g).wait()

    def scatter_issue(s):
        pltpu.async_copy(rows[s], acc.at[didx[s]], sem_s, add=True)

    def scatter_drain(s):
        pltpu.make_async_copy(zeros_hbm.at[pl.ds(0, CHUNK)], rows[s],
                              sem_s).wait()

    idx_issue(0, 0)
    idx_issue(1, 1)
    idx_issue(2, 2)

    base = sid * RPT

    @pl.when(sid < NS - 1)
    def _():
        pltpu.sync_copy(zeros_hbm.at[pl.ds(base, RPT)],
                        acc.at[pl.ds(base, RPT)])

    @pl.when(sid == NS - 1)
    def _():
        pltpu.sync_copy(zeros_hbm.at[pl.ds(base, RPT_LAST)],
                        acc.at[pl.ds(base, RPT_LAST)])

    plsc.subcore_barrier()
    idx_drain(0)
    gather_issue(0)

    def body(g, carry):
        for b in range(NSET):
            j = g * NSET + b
            sc = b
            sn = (b + 1) % NSET
            sf = (b + 2) % NSET
            gather_drain(sc)
            scatter_issue(sc)

            @pl.when(j >= 1)
            def _():
                scatter_drain(sn)

            @pl.when(jnp.logical_and(j >= 1, j + 2 < TPT))
            def _():
                idx_issue_j = j + 2
                pltpu.async_copy(srcb_hbm.at[wid, idx_issue_j], sidx[sf],
                                 sem_i)
                pltpu.async_copy(dstb_hbm.at[wid, idx_issue_j], didx[sf],
                                 sem_i)

            @pl.when(j + 1 < TPT)
            def _():
                idx_drain(sn)
                gather_issue(sn)

        return carry

    lax.fori_loop(0, TPT // NSET, body, 0)

    scatter_drain((TPT - 1) % NSET)
    plsc.subcore_barrier()

    @pl.when(sid < NS - 1)
    def _():
        pltpu.sync_copy(acc.at[pl.ds(base, RPT)],
                        out_hbm.at[pl.ds(cid * N + base, RPT)])

    @pl.when(sid == NS - 1)
    def _():
        pltpu.sync_copy(acc.at[pl.ds(base, RPT_LAST)],
                        out_hbm.at[pl.ds(cid * N + base, RPT_LAST)])


_seg_sum = pl.kernel(
    _seg_sum_body,
    out_type=jax.ShapeDtypeStruct((NC * N, D), jnp.float32),
    mesh=plsc.VectorSubcoreMesh(core_axis_name="c", subcore_axis_name="s"),
    scratch_types=[
        pltpu.VMEM((CHUNK,), jnp.int32),
        pltpu.VMEM((CHUNK,), jnp.int32),
        pltpu.VMEM((CHUNK,), jnp.int32),
        pltpu.VMEM((CHUNK,), jnp.int32),
        pltpu.VMEM((CHUNK,), jnp.int32),
        pltpu.VMEM((CHUNK,), jnp.int32),
        pltpu.VMEM((CHUNK, D), jnp.float32),
        pltpu.VMEM((CHUNK, D), jnp.float32),
        pltpu.VMEM((CHUNK, D), jnp.float32),
        pltpu.VMEM_SHARED((ACC_ROWS, D), jnp.float32),
        pltpu.SemaphoreType.DMA,
        pltpu.SemaphoreType.DMA,
        pltpu.SemaphoreType.DMA,
    ],
)


def _bn(a, g, b):
    m = jnp.mean(a, axis=0)
    v = jnp.mean((a - m) * (a - m), axis=0)
    return (a - m) * lax.rsqrt(v + BN_EPS) * g + b


def _dense_body(h_ref, part_ref, w1_ref, b1_ref, gi_ref, bi_ref,
                w2_ref, b2_ref, go_ref, bo_ref, o_ref, *, relu_out):
    s = h_ref[...] + part_ref[:N] + part_ref[N:]
    a = jnp.dot(s, w1_ref[...], preferred_element_type=jnp.float32)
    a = a + b1_ref[...]
    a = jnp.maximum(_bn(a, gi_ref[...], bi_ref[...]), 0.0)
    o = jnp.dot(a, w2_ref[...], preferred_element_type=jnp.float32)
    o = o + b2_ref[...]
    o = _bn(o, go_ref[...], bo_ref[...])
    if relu_out:
        o = jnp.maximum(o, 0.0)
    o_ref[...] = o


def _dense(h, part, w1, b1, gi, bi, w2, b2, go, bo, relu_out):
    return pl.pallas_call(
        functools.partial(_dense_body, relu_out=relu_out),
        out_shape=jax.ShapeDtypeStruct((N, D), jnp.float32),
    )(h, part, w1, b1, gi, bi, w2, b2, go, bo)


def kernel(x, edge_index, batch,
           w1_0, b1_0, gi_0, bi_0, w2_0, b2_0, go_0, bo_0,
           w1_1, b1_1, gi_1, bi_1, w2_1, b2_1, go_1, bo_1,
           w1_2, b1_2, gi_2, bi_2, w2_2, b2_2, go_2, bo_2):
    src = edge_index[0]
    dst = edge_index[1]
    pad = E_PAD - E
    srcb = jnp.concatenate([src, jnp.zeros((pad,), jnp.int32)])
    srcb = srcb.reshape(NW, TPT, CHUNK)
    dstb = jnp.concatenate([dst, jnp.full((pad,), N, jnp.int32)])
    dstb = dstb.reshape(NW, TPT, CHUNK)
    zeros = jnp.zeros((N, D), jnp.float32)

    params = [
        (w1_0, b1_0, gi_0, bi_0, w2_0, b2_0, go_0, bo_0),
        (w1_1, b1_1, gi_1, bi_1, w2_1, b2_1, go_1, bo_1),
        (w1_2, b1_2, gi_2, bi_2, w2_2, b2_2, go_2, bo_2),
    ]

    h = x
    for l in range(3):
        part = _seg_sum(h, srcb, dstb, zeros)
        h = _dense(h, part, *params[l], relu_out=(l < 2))
    return h


# speedup vs baseline: 1.3252x; 1.3252x over previous
import functools

import jax
import jax.numpy as jnp
from jax import lax
from jax.experimental import pallas as pl
from jax.experimental.pallas import tpu as pltpu
from jax.experimental.pallas import tpu_sc as plsc

N = 10000
E = 320000
D = 128
BN_EPS = 1e-5

NC = 2
NS = 16
NW = NC * NS

CHUNK = 128
NSET = 2
TPT = 80
E_PAD = NW * TPT * CHUNK
ACC_ROWS = N + 8

RPT = 624
RPT_LAST = N - 15 * RPT


def _seg_sum_body(h_hbm, srcb_hbm, dstb_hbm, zeros_hbm, out_hbm,
                  si0, si1, di0, di1, rows, acc, sem_i, sem_g):
    cid = lax.axis_index("c")
    sid = lax.axis_index("s")
    wid = cid * NS + sid
    sidx = [si0, si1]
    didx = [di0, di1]

    def idx_issue(j, s):
        pltpu.async_copy(srcb_hbm.at[wid, j], sidx[s], sem_i)
        pltpu.async_copy(dstb_hbm.at[wid, j], didx[s], sem_i)

    def idx_drain(s):
        pltpu.make_async_copy(dstb_hbm.at[wid, 0], sidx[s], sem_i).wait()
        pltpu.make_async_copy(dstb_hbm.at[wid, 0], didx[s], sem_i).wait()

    idx_issue(0, 0)
    idx_issue(1, 1)

    base = sid * RPT

    @pl.when(sid < NS - 1)
    def _():
        pltpu.sync_copy(zeros_hbm.at[pl.ds(base, RPT)],
                        acc.at[pl.ds(base, RPT)])

    @pl.when(sid == NS - 1)
    def _():
        pltpu.sync_copy(zeros_hbm.at[pl.ds(base, RPT_LAST)],
                        acc.at[pl.ds(base, RPT_LAST)])

    plsc.subcore_barrier()

    def body(g, carry):
        for b in range(NSET):
            j = g * NSET + b
            idx_drain(b)
            pltpu.async_copy(h_hbm.at[sidx[b]], rows, sem_g).wait()
            pltpu.sync_copy(rows, acc.at[didx[b]], add=True)

            @pl.when(j + NSET < TPT)
            def _():
                pltpu.async_copy(srcb_hbm.at[wid, j + NSET], sidx[b], sem_i)
                pltpu.async_copy(dstb_hbm.at[wid, j + NSET], didx[b], sem_i)

        return carry

    lax.fori_loop(0, TPT // NSET, body, 0)
    plsc.subcore_barrier()

    @pl.when(sid < NS - 1)
    def _():
        pltpu.sync_copy(acc.at[pl.ds(base, RPT)],
                        out_hbm.at[pl.ds(cid * N + base, RPT)])

    @pl.when(sid == NS - 1)
    def _():
        pltpu.sync_copy(acc.at[pl.ds(base, RPT_LAST)],
                        out_hbm.at[pl.ds(cid * N + base, RPT_LAST)])


_seg_sum = pl.kernel(
    _seg_sum_body,
    out_type=jax.ShapeDtypeStruct((NC * N, D), jnp.float32),
    mesh=plsc.VectorSubcoreMesh(core_axis_name="c", subcore_axis_name="s"),
    scratch_types=[
        pltpu.VMEM((CHUNK,), jnp.int32),
        pltpu.VMEM((CHUNK,), jnp.int32),
        pltpu.VMEM((CHUNK,), jnp.int32),
        pltpu.VMEM((CHUNK,), jnp.int32),
        pltpu.VMEM((CHUNK, D), jnp.float32),
        pltpu.VMEM_SHARED((ACC_ROWS, D), jnp.float32),
        pltpu.SemaphoreType.DMA,
        pltpu.SemaphoreType.DMA,
    ],
)


def _bn(a, g, b):
    m = jnp.mean(a, axis=0)
    v = jnp.mean((a - m) * (a - m), axis=0)
    return (a - m) * lax.rsqrt(v + BN_EPS) * g + b


def _dense_body(h_ref, part_ref, w1_ref, b1_ref, gi_ref, bi_ref,
                w2_ref, b2_ref, go_ref, bo_ref, o_ref, *, relu_out):
    s = h_ref[...] + part_ref[:N] + part_ref[N:]
    a = jnp.dot(s, w1_ref[...], preferred_element_type=jnp.float32)
    a = a + b1_ref[...]
    a = jnp.maximum(_bn(a, gi_ref[...], bi_ref[...]), 0.0)
    o = jnp.dot(a, w2_ref[...], preferred_element_type=jnp.float32)
    o = o + b2_ref[...]
    o = _bn(o, go_ref[...], bo_ref[...])
    if relu_out:
        o = jnp.maximum(o, 0.0)
    o_ref[...] = o


def _dense(h, part, w1, b1, gi, bi, w2, b2, go, bo, relu_out):
    return pl.pallas_call(
        functools.partial(_dense_body, relu_out=relu_out),
        out_shape=jax.ShapeDtypeStruct((N, D), jnp.float32),
    )(h, part, w1, b1, gi, bi, w2, b2, go, bo)


def kernel(x, edge_index, batch,
           w1_0, b1_0, gi_0, bi_0, w2_0, b2_0, go_0, bo_0,
           w1_1, b1_1, gi_1, bi_1, w2_1, b2_1, go_1, bo_1,
           w1_2, b1_2, gi_2, bi_2, w2_2, b2_2, go_2, bo_2):
    src = edge_index[0]
    dst = edge_index[1]
    pad = E_PAD - E
    srcb = jnp.concatenate([src, jnp.zeros((pad,), jnp.int32)])
    srcb = srcb.reshape(NW, TPT, CHUNK)
    dstb = jnp.concatenate([dst, jnp.full((pad,), N, jnp.int32)])
    dstb = dstb.reshape(NW, TPT, CHUNK)
    zeros = jnp.zeros((N, D), jnp.float32)

    params = [
        (w1_0, b1_0, gi_0, bi_0, w2_0, b2_0, go_0, bo_0),
        (w1_1, b1_1, gi_1, bi_1, w2_1, b2_1, go_1, bo_1),
        (w1_2, b1_2, gi_2, bi_2, w2_2, b2_2, go_2, bo_2),
    ]

    h = x
    for l in range(3):
        part = _seg_sum(h, srcb, dstb, zeros)
        h = _dense(h, part, *params[l], relu_out=(l < 2))
    return h
